# baseline (speedup 1.0000x reference)
import functools

import jax
import jax.numpy as jnp
from jax import lax
from jax.experimental import pallas as pl
from jax.experimental.pallas import tpu as pltpu

Q = 16
DIM = 64
PACK = 4
KTOP = 10
NSEL = 16
NCAND = 48
BUF = 128
EPS = 0.05


def _fold_slots(x):
    out = x[:, 0:Q]
    for s in range(1, PACK):
        out = jnp.minimum(out, x[:, s * Q:(s + 1) * Q])
    return out


def _knn_kernel(a1_ref, a2_ref, k_ref, dout_ref, iout_ref, dscr_ref,
                *, block_k):
    t = pl.program_id(0)
    rows = block_k // PACK

    @pl.when(t == 0)
    def _init():
        dout_ref[...] = jnp.full((BUF, Q), jnp.inf, jnp.float32)
        iout_ref[...] = jnp.zeros((BUF, Q), jnp.int32)

    kb = jnp.concatenate(
        [k_ref[s * rows:(s + 1) * rows, :] for s in range(PACK)],
        axis=1)
    a1 = a1_ref[...]
    a2 = a2_ref[...]

    qk = lax.dot_general(kb, a1, (((1,), (0,)), ((), ())),
                         preferred_element_type=jnp.float32,
                         precision=lax.Precision.HIGHEST)
    c2 = lax.dot_general(kb * kb, a2, (((1,), (0,)), ((), ())),
                         preferred_element_type=jnp.float32,
                         precision=lax.Precision.HIGHEST)
    d = c2 + qk

    rowi = lax.broadcasted_iota(jnp.int32, (rows, PACK * Q), 0)
    slot = lax.broadcasted_iota(jnp.int32, (rows, PACK * Q), 1) // Q
    base = (t * block_k).astype(jnp.int32)
    gidx = slot * rows + rowi + base
    bufi = lax.broadcasted_iota(jnp.int32, (BUF, Q), 0)
    BIGI = jnp.int32(2**31 - 1)

    tau = dout_ref[KTOP - 1:KTOP, :]
    tau4 = jnp.concatenate([tau] * PACK, axis=1)
    hit = jnp.any(d < tau4 + EPS)

    @pl.when(hit)
    def _merge():
        dscr_ref[...] = d

        def cond(c):
            return (c[0] < NSEL) & c[1]

        def body(c):
            r, _ = c
            dd = dscr_ref[...]
            mcol = jnp.min(dd, axis=0, keepdims=True)
            mq = _fold_slots(mcol)
            mq4 = jnp.concatenate([mq] * PACK, axis=1)
            g = jnp.min(jnp.where(dd == mq4, gidx, BIGI),
                        axis=0, keepdims=True)
            gq = _fold_slots(g)
            gq4 = jnp.concatenate([gq] * PACK, axis=1)
            dscr_ref[...] = jnp.where(gidx == gq4, jnp.inf, dd)

            vals = dout_ref[...]
            idxs = iout_ref[...]
            do_q = mq < vals[KTOP - 1:KTOP, :] + EPS
            pos = jnp.sum((vals <= mq).astype(jnp.int32),
                          axis=0, keepdims=True)
            vshift = jnp.concatenate([vals[:1], vals[:-1]], axis=0)
            ishift = jnp.concatenate([idxs[:1], idxs[:-1]], axis=0)
            newv = jnp.where(bufi < pos, vals,
                             jnp.where(bufi == pos, mq, vshift))
            newi = jnp.where(bufi < pos, idxs,
                             jnp.where(bufi == pos, gq, ishift))
            dout_ref[...] = jnp.where(do_q, newv, vals)
            iout_ref[...] = jnp.where(do_q, newi, idxs)
            return r + jnp.int32(1), jnp.any(do_q)

        lax.while_loop(cond, body, (jnp.int32(0), True))


def kernel(queries, keys, k):
    nkeys = keys.shape[0]
    block_k = 20000
    assert nkeys % block_k == 0
    nb = nkeys // block_k
    rows = block_k // PACK

    eye = jnp.eye(PACK, dtype=jnp.float32)
    a1 = jnp.einsum("st,dq->sdtq", eye, -2.0 * queries.T).reshape(
        PACK * DIM, PACK * Q)
    a2 = jnp.einsum("st,dq->sdtq", eye,
                    jnp.ones((DIM, Q), jnp.float32)).reshape(
        PACK * DIM, PACK * Q)

    _, ipad = pl.pallas_call(
        functools.partial(_knn_kernel, block_k=block_k),
        grid=(nb,),
        in_specs=[
            pl.BlockSpec((PACK * DIM, PACK * Q), lambda t: (0, 0)),
            pl.BlockSpec((PACK * DIM, PACK * Q), lambda t: (0, 0)),
            pl.BlockSpec((block_k, DIM), lambda t: (t, 0)),
        ],
        out_specs=[
            pl.BlockSpec((BUF, Q), lambda t: (0, 0)),
            pl.BlockSpec((BUF, Q), lambda t: (0, 0)),
        ],
        out_shape=[
            jax.ShapeDtypeStruct((BUF, Q), jnp.float32),
            jax.ShapeDtypeStruct((BUF, Q), jnp.int32),
        ],
        scratch_shapes=[pltpu.VMEM((rows, PACK * Q), jnp.float32)],
    )(a1, a2, keys)

    cand = jnp.sort(ipad[:NCAND, :].reshape(-1))
    dup = jnp.concatenate(
        [jnp.zeros((1,), jnp.bool_), cand[1:] == cand[:-1]])
    sub = keys[cand]
    q2 = jnp.sum(queries * queries, axis=1, keepdims=True)
    c2 = jnp.sum(sub * sub, axis=1)[None, :]
    d2 = q2 + c2 - 2.0 * (queries @ sub.T)
    d2 = jnp.where(dup[None, :], jnp.inf, d2)
    neg_vals, pos = lax.top_k(-d2, KTOP)
    D = -neg_vals
    I = cand[pos]
    kth = D[-1, -1]
    return D, I, kth


# speedup vs baseline: 1.3821x; 1.3821x over previous
import functools

import jax
import jax.numpy as jnp
from jax import lax
from jax.experimental import pallas as pl
from jax.experimental.pallas import tpu as pltpu

Q = 16
DIM = 64
PACK = 4
KTOP = 10
NSEL = 16
NCAND = 48
BUF = 128
EPS = 1.0


def _fold_slots(x):
    out = x[:, 0:Q]
    for s in range(1, PACK):
        out = jnp.minimum(out, x[:, s * Q:(s + 1) * Q])
    return out


def _knn_kernel(a1_ref, a2_ref, k_ref, dout_ref, iout_ref, dscr_ref,
                *, block_k):
    t = pl.program_id(0)
    rows = block_k // PACK

    @pl.when(t == 0)
    def _init():
        dout_ref[...] = jnp.full((BUF, Q), jnp.inf, jnp.float32)
        iout_ref[...] = jnp.zeros((BUF, Q), jnp.int32)

    kb = jnp.concatenate(
        [k_ref[s * rows:(s + 1) * rows, :] for s in range(PACK)],
        axis=1)
    a1 = a1_ref[...]
    a2 = a2_ref[...]

    kbb = kb.astype(jnp.bfloat16)
    ksqb = (kb * kb).astype(jnp.bfloat16)
    qk = lax.dot_general(kbb, a1, (((1,), (0,)), ((), ())),
                         preferred_element_type=jnp.float32)
    c2 = lax.dot_general(ksqb, a2, (((1,), (0,)), ((), ())),
                         preferred_element_type=jnp.float32)
    d = c2 + qk

    rowi = lax.broadcasted_iota(jnp.int32, (rows, PACK * Q), 0)
    slot = lax.broadcasted_iota(jnp.int32, (rows, PACK * Q), 1) // Q
    base = (t * block_k).astype(jnp.int32)
    gidx = slot * rows + rowi + base
    bufi = lax.broadcasted_iota(jnp.int32, (BUF, Q), 0)
    BIGI = jnp.int32(2**31 - 1)

    tau = dout_ref[KTOP - 1:KTOP, :]
    tau4 = jnp.concatenate([tau] * PACK, axis=1)
    hit = jnp.any(d < tau4 + EPS)

    @pl.when(hit)
    def _merge():
        dscr_ref[...] = d

        def cond(c):
            return (c[0] < NSEL) & c[1]

        def body(c):
            r, _ = c
            dd = dscr_ref[...]
            mcol = jnp.min(dd, axis=0, keepdims=True)
            mq = _fold_slots(mcol)
            mq4 = jnp.concatenate([mq] * PACK, axis=1)
            g = jnp.min(jnp.where(dd == mq4, gidx, BIGI),
                        axis=0, keepdims=True)
            gq = _fold_slots(g)
            gq4 = jnp.concatenate([gq] * PACK, axis=1)
            dscr_ref[...] = jnp.where(gidx == gq4, jnp.inf, dd)

            vals = dout_ref[...]
            idxs = iout_ref[...]
            do_q = mq < vals[KTOP - 1:KTOP, :] + EPS
            pos = jnp.sum((vals <= mq).astype(jnp.int32),
                          axis=0, keepdims=True)
            vshift = jnp.concatenate([vals[:1], vals[:-1]], axis=0)
            ishift = jnp.concatenate([idxs[:1], idxs[:-1]], axis=0)
            newv = jnp.where(bufi < pos, vals,
                             jnp.where(bufi == pos, mq, vshift))
            newi = jnp.where(bufi < pos, idxs,
                             jnp.where(bufi == pos, gq, ishift))
            dout_ref[...] = jnp.where(do_q, newv, vals)
            iout_ref[...] = jnp.where(do_q, newi, idxs)
            return r + jnp.int32(1), jnp.any(do_q)

        lax.while_loop(cond, body, (jnp.int32(0), True))


def kernel(queries, keys, k):
    nkeys = keys.shape[0]
    block_k = 20000
    assert nkeys % block_k == 0
    nb = nkeys // block_k
    rows = block_k // PACK

    eye = jnp.eye(PACK, dtype=jnp.float32)
    a1 = jnp.einsum("st,dq->sdtq", eye, -2.0 * queries.T).reshape(
        PACK * DIM, PACK * Q).astype(jnp.bfloat16)
    a2 = jnp.einsum("st,dq->sdtq", eye,
                    jnp.ones((DIM, Q), jnp.float32)).reshape(
        PACK * DIM, PACK * Q).astype(jnp.bfloat16)

    _, ipad = pl.pallas_call(
        functools.partial(_knn_kernel, block_k=block_k),
        grid=(nb,),
        in_specs=[
            pl.BlockSpec((PACK * DIM, PACK * Q), lambda t: (0, 0)),
            pl.BlockSpec((PACK * DIM, PACK * Q), lambda t: (0, 0)),
            pl.BlockSpec((block_k, DIM), lambda t: (t, 0)),
        ],
        out_specs=[
            pl.BlockSpec((BUF, Q), lambda t: (0, 0)),
            pl.BlockSpec((BUF, Q), lambda t: (0, 0)),
        ],
        out_shape=[
            jax.ShapeDtypeStruct((BUF, Q), jnp.float32),
            jax.ShapeDtypeStruct((BUF, Q), jnp.int32),
        ],
        scratch_shapes=[pltpu.VMEM((rows, PACK * Q), jnp.float32)],
    )(a1, a2, keys)

    cand = jnp.sort(ipad[:NCAND, :].reshape(-1))
    dup = jnp.concatenate(
        [jnp.zeros((1,), jnp.bool_), cand[1:] == cand[:-1]])
    sub = keys[cand]
    q2 = jnp.sum(queries * queries, axis=1, keepdims=True)
    c2 = jnp.sum(sub * sub, axis=1)[None, :]
    d2 = q2 + c2 - 2.0 * (queries @ sub.T)
    d2 = jnp.where(dup[None, :], jnp.inf, d2)
    neg_vals, pos = lax.top_k(-d2, KTOP)
    D = -neg_vals
    I = cand[pos]
    kth = D[-1, -1]
    return D, I, kth
